# baseline (speedup 1.0000x reference)
import functools

import jax
import jax.numpy as jnp
from jax import lax
from jax.experimental import pallas as pl
from jax.experimental.pallas import tpu as pltpu
from jax.experimental.pallas import tpu_sc as plsc

N = 10000
E = 320000
D_IN = 128
H = 48
G = 64
OUT = 1

NC = 2
NS = 16
NW = NC * NS
EPW = E // NW
K = 80
C = EPW // K
ROWS_PT = N // NS

_MESH = plsc.VectorSubcoreMesh(
    core_axis_name="c", subcore_axis_name="s", num_cores=NC, num_subcores=NS
)

_SC_PARAMS = pltpu.CompilerParams(use_tc_tiling_on_sc=False)


@functools.partial(
    pl.kernel,
    out_type=jax.ShapeDtypeStruct((NC, N), jnp.float32),
    mesh=_MESH,
    compiler_params=_SC_PARAMS,
    scratch_types=[
        pltpu.VMEM((C, K), jnp.int32),
        pltpu.VMEM((K,), jnp.float32),
        pltpu.VMEM_SHARED((N,), jnp.float32),
        pltpu.SemaphoreType.DMA,
    ],
)
def _sc_degree(ei_hbm, zeros_hbm, out_hbm, dst_v, ones_v, deg_sh, ssem):
    cid = lax.axis_index("c")
    sid = lax.axis_index("s")
    wid = sid * NC + cid
    pltpu.sync_copy(ei_hbm.at[1, wid], dst_v)
    for i in range(K // 16):
        ones_v[pl.ds(i * 16, 16)] = jnp.ones((16,), jnp.float32)

    @pl.when(sid < 10)
    def _():
        pltpu.sync_copy(
            zeros_hbm.at[pl.ds(sid * 1000, 1000)],
            deg_sh.at[pl.ds(sid * 1000, 1000)],
        )

    plsc.subcore_barrier()

    def body(o, carry):
        base = 8 * o
        for t in range(8):
            pltpu.async_copy(ones_v, deg_sh.at[dst_v.at[base + t]], ssem,
                             add=True)
        for t in range(8):
            pltpu.make_async_copy(ones_v, deg_sh.at[dst_v.at[base + t]],
                                  ssem).wait()
        return carry

    lax.fori_loop(0, C // 8, body, 0)
    for t in range(C - (C // 8) * 8):
        pltpu.async_copy(ones_v, deg_sh.at[dst_v.at[(C // 8) * 8 + t]], ssem,
                         add=True)
    for t in range(C - (C // 8) * 8):
        pltpu.make_async_copy(ones_v, deg_sh.at[dst_v.at[(C // 8) * 8 + t]],
                              ssem).wait()
    plsc.subcore_barrier()

    @pl.when(sid < 10)
    def _():
        pltpu.sync_copy(
            deg_sh.at[pl.ds(sid * 1000, 1000)],
            out_hbm.at[cid, pl.ds(sid * 1000, 1000)],
        )


NBUF = 6
PRE = 3


@functools.partial(
    pl.kernel,
    out_type=jax.ShapeDtypeStruct((NC, N, H), jnp.float32),
    mesh=_MESH,
    compiler_params=_SC_PARAMS,
    scratch_types=[
        pltpu.VMEM((2, C, K), jnp.int32),
        pltpu.VMEM((NBUF, K, H), jnp.float32),
        pltpu.VMEM_SHARED((N, H), jnp.float32),
        pltpu.VMEM_SHARED((N, H), jnp.float32),
        pltpu.SemaphoreType.DMA((NBUF,)),
        pltpu.SemaphoreType.DMA((NBUF,)),
        pltpu.SemaphoreType.DMA,
    ],
)
def _sc_conv(ei_hbm, u_hbm, out_hbm, idx_v, buf, u_sh, s_sh, gsem, ssem, isem):
    cid = lax.axis_index("c")
    sid = lax.axis_index("s")
    wid = sid * NC + cid
    pltpu.async_copy(ei_hbm.at[0, wid], idx_v.at[0], isem)
    pltpu.async_copy(ei_hbm.at[1, wid], idx_v.at[1], isem)
    rows = pl.ds(sid * ROWS_PT, ROWS_PT)

    pltpu.async_copy(u_hbm.at[rows], u_sh.at[rows], isem)
    pltpu.sync_copy(u_hbm.at[rows], s_sh.at[rows])
    pltpu.make_async_copy(ei_hbm.at[0, wid], idx_v.at[0], isem).wait()
    pltpu.make_async_copy(ei_hbm.at[1, wid], idx_v.at[1], isem).wait()
    pltpu.make_async_copy(u_hbm.at[rows], u_sh.at[rows], isem).wait()
    plsc.subcore_barrier()

    def fire_gather(j, b):
        pltpu.async_copy(u_sh.at[idx_v.at[0, j]], buf.at[b], gsem.at[b])

    def fire_scatter(i, b):
        pltpu.async_copy(buf.at[b], s_sh.at[idx_v.at[1, i]], ssem.at[b],
                         add=True)

    def wait_scatter(i, b):
        pltpu.make_async_copy(buf.at[b], s_sh.at[idx_v.at[1, i]],
                              ssem.at[b]).wait()

    for j in range(PRE):
        fire_gather(j, j)

    def body(i, carry):
        j = i + PRE
        b = lax.rem(i, NBUF)
        bj = lax.rem(j, NBUF)

        @pl.when(j < C)
        def _():
            @pl.when(j >= NBUF)
            def _():
                wait_scatter(j - NBUF, bj)
            fire_gather(j, bj)

        pltpu.make_async_copy(u_sh.at[idx_v.at[0, i]],
                              buf.at[b], gsem.at[b]).wait()
        fire_scatter(i, b)
        return carry

    lax.fori_loop(0, C, body, 0)
    for t in range(NBUF):
        i = C - NBUF + t
        wait_scatter(i, i % NBUF)

    plsc.subcore_barrier()
    pltpu.sync_copy(s_sh.at[rows], out_hbm.at[cid, rows])


_INV48 = 1.0 / 48.0


def _row_sum48(a):
    s = a[:, 0:8]
    for k in range(1, 6):
        s = s + a[:, 8 * k:8 * k + 8]
    s = s[:, 0:4] + s[:, 4:8]
    s = s[:, 0:2] + s[:, 2:4]
    return s[:, 0:1] + s[:, 1:2]


def _tc_encoder_body(x_ref, we_ref, be_ref, lg_ref, lb_ref, wc1_ref,
                     d0_ref, d1_ref, u1_ref, dis_ref):
    h = jnp.maximum(x_ref[...] @ we_ref[...] + be_ref[...], 0.0)
    m = _row_sum48(h) * _INV48
    hc = h - m
    v = _row_sum48(hc * hc) * _INV48
    h = hc / jnp.sqrt(v + 1e-5) * lg_ref[...] + lb_ref[...]
    deg = d0_ref[...] + d1_ref[...] + 1.0
    dis = lax.rsqrt(deg)
    dis_ref[...] = dis
    u1_ref[...] = (h @ wc1_ref[...]) * dis


def _encode(x, W_enc, b_enc, ln_g, ln_b, W_c1, deg0, deg1):
    NB = 1000
    return pl.pallas_call(
        _tc_encoder_body,
        grid=(N // NB,),
        in_specs=[
            pl.BlockSpec((NB, D_IN), lambda i: (i, 0)),
            pl.BlockSpec((D_IN, H), lambda i: (0, 0)),
            pl.BlockSpec((H,), lambda i: (0,)),
            pl.BlockSpec((H,), lambda i: (0,)),
            pl.BlockSpec((H,), lambda i: (0,)),
            pl.BlockSpec((H, H), lambda i: (0, 0)),
            pl.BlockSpec((NB, 1), lambda i: (i, 0)),
            pl.BlockSpec((NB, 1), lambda i: (i, 0)),
        ],
        out_specs=(pl.BlockSpec((NB, H), lambda i: (i, 0)),
                   pl.BlockSpec((NB, 1), lambda i: (i, 0))),
        out_shape=(jax.ShapeDtypeStruct((N, H), jnp.float32),
                   jax.ShapeDtypeStruct((N, 1), jnp.float32)),
    )(x, W_enc, b_enc, ln_g, ln_b, W_c1, deg0, deg1)


def _tc_mid_body(s_ref, u_ref, dis_ref, bc1_ref, wc2_ref, u2_ref):
    dis = dis_ref[...]
    h1 = jnp.maximum((s_ref[0] + s_ref[1] - u_ref[...]) * dis + bc1_ref[...],
                     0.0)
    u2_ref[...] = (h1 @ wc2_ref[...]) * dis


def _tc_final_body(s_ref, u_ref, dis_ref, bc2_ref, batch_ref,
                   w1_ref, b1_ref, w2_ref, b2_ref, out_ref):
    dis = dis_ref[...]
    h2 = jnp.maximum((s_ref[0] + s_ref[1] - u_ref[...]) * dis + bc2_ref[...],
                     0.0)
    b = batch_ref[...]
    gids = lax.broadcasted_iota(jnp.int32, (G, N), 0)
    m = (gids == b[None, :]).astype(jnp.float32)
    sums = jnp.dot(m, h2, preferred_element_type=jnp.float32)
    cnt = jnp.sum(m, axis=1, keepdims=True)
    pooled = sums / jnp.maximum(cnt, 1.0)
    hm = jnp.maximum(jnp.dot(pooled, w1_ref[...],
                             preferred_element_type=jnp.float32) + b1_ref[...],
                     0.0)
    out_ref[...] = jnp.dot(hm, w2_ref[...],
                           preferred_element_type=jnp.float32) + b2_ref[...]


def kernel(x, edge_index, batch, W_enc, b_enc, ln_g, ln_b,
           W_c1, b_c1, W_c2, b_c2, W1, b1, W2, b2):
    ei4 = edge_index.reshape(2, NW, C, K)
    zeros1 = jnp.zeros((N,), jnp.float32)

    degp = _sc_degree(ei4, zeros1)
    deg0 = degp[0].reshape(N, 1)
    deg1 = degp[1].reshape(N, 1)

    u1, dis = _encode(x, W_enc, b_enc, ln_g, ln_b, W_c1, deg0, deg1)

    S1 = _sc_conv(ei4, u1)

    u2 = pl.pallas_call(
        _tc_mid_body,
        out_shape=jax.ShapeDtypeStruct((N, H), jnp.float32),
    )(S1, u1, dis, b_c1, W_c2)

    S2 = _sc_conv(ei4, u2)

    out = pl.pallas_call(
        _tc_final_body,
        out_shape=jax.ShapeDtypeStruct((G, OUT), jnp.float32),
    )(S2, u2, dis, b_c2, batch, W1, b1, W2, b2)
    return out


# speedup vs baseline: 1.1049x; 1.1049x over previous
import functools

import jax
import jax.numpy as jnp
from jax import lax
from jax.experimental import pallas as pl
from jax.experimental.pallas import tpu as pltpu
from jax.experimental.pallas import tpu_sc as plsc

N = 10000
E = 320000
D_IN = 128
H = 48
G = 64
OUT = 1

NC = 2
NS = 16
NW = NC * NS
EPW = E // NW
K = 80
C = EPW // K
ROWS_PT = N // NS

_MESH = plsc.VectorSubcoreMesh(
    core_axis_name="c", subcore_axis_name="s", num_cores=NC, num_subcores=NS
)

_SC_PARAMS = pltpu.CompilerParams(use_tc_tiling_on_sc=False)


@functools.partial(
    pl.kernel,
    out_type=jax.ShapeDtypeStruct((NC, N), jnp.float32),
    mesh=_MESH,
    compiler_params=_SC_PARAMS,
    scratch_types=[
        pltpu.VMEM((C, K), jnp.int32),
        pltpu.VMEM((K,), jnp.float32),
        pltpu.VMEM_SHARED((N,), jnp.float32),
        pltpu.SemaphoreType.DMA,
    ],
)
def _sc_degree(ei_hbm, zeros_hbm, out_hbm, dst_v, ones_v, deg_sh, ssem):
    cid = lax.axis_index("c")
    sid = lax.axis_index("s")
    wid = sid * NC + cid
    pltpu.sync_copy(ei_hbm.at[1, wid], dst_v)
    for i in range(K // 16):
        ones_v[pl.ds(i * 16, 16)] = jnp.ones((16,), jnp.float32)

    @pl.when(sid < 10)
    def _():
        pltpu.sync_copy(
            zeros_hbm.at[pl.ds(sid * 1000, 1000)],
            deg_sh.at[pl.ds(sid * 1000, 1000)],
        )

    plsc.subcore_barrier()

    def body(o, carry):
        base = 8 * o
        for t in range(8):
            pltpu.async_copy(ones_v, deg_sh.at[dst_v.at[base + t]], ssem,
                             add=True)
        for t in range(8):
            pltpu.make_async_copy(ones_v, deg_sh.at[dst_v.at[base + t]],
                                  ssem).wait()
        return carry

    lax.fori_loop(0, C // 8, body, 0)
    for t in range(C - (C // 8) * 8):
        pltpu.async_copy(ones_v, deg_sh.at[dst_v.at[(C // 8) * 8 + t]], ssem,
                         add=True)
    for t in range(C - (C // 8) * 8):
        pltpu.make_async_copy(ones_v, deg_sh.at[dst_v.at[(C // 8) * 8 + t]],
                              ssem).wait()
    plsc.subcore_barrier()

    @pl.when(sid < 10)
    def _():
        pltpu.sync_copy(
            deg_sh.at[pl.ds(sid * 1000, 1000)],
            out_hbm.at[cid, pl.ds(sid * 1000, 1000)],
        )


NBUF = 8
PRE = 5


@functools.partial(
    pl.kernel,
    out_type=jax.ShapeDtypeStruct((NC, N, H), jnp.float32),
    mesh=_MESH,
    compiler_params=_SC_PARAMS,
    scratch_types=[
        pltpu.VMEM((2, C, K), jnp.int32),
        pltpu.VMEM((NBUF, K, H), jnp.float32),
        pltpu.VMEM_SHARED((N, H), jnp.float32),
        pltpu.SemaphoreType.DMA((NBUF,)),
        pltpu.SemaphoreType.DMA((NBUF,)),
        pltpu.SemaphoreType.DMA,
    ],
)
def _sc_conv(ei_hbm, u_hbm, out_hbm, idx_v, buf, s_sh, gsem, ssem, isem):
    cid = lax.axis_index("c")
    sid = lax.axis_index("s")
    wid = sid * NC + cid
    pltpu.async_copy(ei_hbm.at[0, wid], idx_v.at[0], isem)
    pltpu.async_copy(ei_hbm.at[1, wid], idx_v.at[1], isem)
    rows = pl.ds(sid * ROWS_PT, ROWS_PT)

    pltpu.sync_copy(u_hbm.at[rows], s_sh.at[rows])
    pltpu.make_async_copy(ei_hbm.at[0, wid], idx_v.at[0], isem).wait()
    pltpu.make_async_copy(ei_hbm.at[1, wid], idx_v.at[1], isem).wait()
    plsc.subcore_barrier()

    def fire_gather(j, b):
        pltpu.async_copy(u_hbm.at[idx_v.at[0, j]], buf.at[b], gsem.at[b])

    def fire_scatter(i, b):
        pltpu.async_copy(buf.at[b], s_sh.at[idx_v.at[1, i]], ssem.at[b],
                         add=True)

    def wait_scatter(i, b):
        pltpu.make_async_copy(buf.at[b], s_sh.at[idx_v.at[1, i]],
                              ssem.at[b]).wait()

    for j in range(PRE):
        fire_gather(j, j)

    def body(i, carry):
        j = i + PRE
        b = lax.rem(i, NBUF)
        bj = lax.rem(j, NBUF)

        @pl.when(j < C)
        def _():
            @pl.when(j >= NBUF)
            def _():
                wait_scatter(j - NBUF, bj)
            fire_gather(j, bj)

        pltpu.make_async_copy(u_hbm.at[idx_v.at[0, i]],
                              buf.at[b], gsem.at[b]).wait()
        fire_scatter(i, b)
        return carry

    lax.fori_loop(0, C, body, 0)
    for t in range(NBUF):
        i = C - NBUF + t
        wait_scatter(i, i % NBUF)

    plsc.subcore_barrier()
    pltpu.sync_copy(s_sh.at[rows], out_hbm.at[cid, rows])


_INV48 = 1.0 / 48.0


def _row_sum48(a):
    s = a[:, 0:8]
    for k in range(1, 6):
        s = s + a[:, 8 * k:8 * k + 8]
    s = s[:, 0:4] + s[:, 4:8]
    s = s[:, 0:2] + s[:, 2:4]
    return s[:, 0:1] + s[:, 1:2]


def _tc_encoder_body(x_ref, we_ref, be_ref, lg_ref, lb_ref, wc1_ref,
                     d0_ref, d1_ref, u1_ref, dis_ref):
    h = jnp.maximum(x_ref[...] @ we_ref[...] + be_ref[...], 0.0)
    m = _row_sum48(h) * _INV48
    hc = h - m
    v = _row_sum48(hc * hc) * _INV48
    h = hc / jnp.sqrt(v + 1e-5) * lg_ref[...] + lb_ref[...]
    deg = d0_ref[...] + d1_ref[...] + 1.0
    dis = lax.rsqrt(deg)
    dis_ref[...] = dis
    u1_ref[...] = (h @ wc1_ref[...]) * dis


def _encode(x, W_enc, b_enc, ln_g, ln_b, W_c1, deg0, deg1):
    NB = 1000
    return pl.pallas_call(
        _tc_encoder_body,
        grid=(N // NB,),
        in_specs=[
            pl.BlockSpec((NB, D_IN), lambda i: (i, 0)),
            pl.BlockSpec((D_IN, H), lambda i: (0, 0)),
            pl.BlockSpec((H,), lambda i: (0,)),
            pl.BlockSpec((H,), lambda i: (0,)),
            pl.BlockSpec((H,), lambda i: (0,)),
            pl.BlockSpec((H, H), lambda i: (0, 0)),
            pl.BlockSpec((NB, 1), lambda i: (i, 0)),
            pl.BlockSpec((NB, 1), lambda i: (i, 0)),
        ],
        out_specs=(pl.BlockSpec((NB, H), lambda i: (i, 0)),
                   pl.BlockSpec((NB, 1), lambda i: (i, 0))),
        out_shape=(jax.ShapeDtypeStruct((N, H), jnp.float32),
                   jax.ShapeDtypeStruct((N, 1), jnp.float32)),
    )(x, W_enc, b_enc, ln_g, ln_b, W_c1, deg0, deg1)


def _tc_mid_body(s_ref, u_ref, dis_ref, bc1_ref, wc2_ref, u2_ref):
    dis = dis_ref[...]
    h1 = jnp.maximum((s_ref[0] + s_ref[1] - u_ref[...]) * dis + bc1_ref[...],
                     0.0)
    u2_ref[...] = (h1 @ wc2_ref[...]) * dis


def _tc_final_body(s_ref, u_ref, dis_ref, bc2_ref, batch_ref,
                   w1_ref, b1_ref, w2_ref, b2_ref, out_ref):
    dis = dis_ref[...]
    h2 = jnp.maximum((s_ref[0] + s_ref[1] - u_ref[...]) * dis + bc2_ref[...],
                     0.0)
    b = batch_ref[...]
    gids = lax.broadcasted_iota(jnp.int32, (G, N), 0)
    m = (gids == b[None, :]).astype(jnp.float32)
    sums = jnp.dot(m, h2, preferred_element_type=jnp.float32)
    cnt = jnp.sum(m, axis=1, keepdims=True)
    pooled = sums / jnp.maximum(cnt, 1.0)
    hm = jnp.maximum(jnp.dot(pooled, w1_ref[...],
                             preferred_element_type=jnp.float32) + b1_ref[...],
                     0.0)
    out_ref[...] = jnp.dot(hm, w2_ref[...],
                           preferred_element_type=jnp.float32) + b2_ref[...]


def kernel(x, edge_index, batch, W_enc, b_enc, ln_g, ln_b,
           W_c1, b_c1, W_c2, b_c2, W1, b1, W2, b2):
    ei4 = edge_index.reshape(2, NW, C, K)
    zeros1 = jnp.zeros((N,), jnp.float32)

    degp = _sc_degree(ei4, zeros1)
    deg0 = degp[0].reshape(N, 1)
    deg1 = degp[1].reshape(N, 1)

    u1, dis = _encode(x, W_enc, b_enc, ln_g, ln_b, W_c1, deg0, deg1)

    S1 = _sc_conv(ei4, u1)

    u2 = pl.pallas_call(
        _tc_mid_body,
        out_shape=jax.ShapeDtypeStruct((N, H), jnp.float32),
    )(S1, u1, dis, b_c1, W_c2)

    S2 = _sc_conv(ei4, u2)

    out = pl.pallas_call(
        _tc_final_body,
        out_shape=jax.ShapeDtypeStruct((G, OUT), jnp.float32),
    )(S2, u2, dis, b_c2, batch, W1, b1, W2, b2)
    return out
